# baseline (speedup 1.0000x reference)
import functools

import jax
import jax.numpy as jnp
from jax import lax
from jax.experimental import pallas as pl
from jax.experimental.pallas import tpu as pltpu
from jax.experimental.pallas import tpu_sc as plsc

VOCAB = 100000
DIM = 64
BATCH = 4096
HIST = 50

NW = 32
BBLK = BATCH // NW
L = 16

_mesh = plsc.VectorSubcoreMesh(core_axis_name="c", subcore_axis_name="s")


@functools.partial(
    pl.kernel,
    mesh=_mesh,
    out_type=jax.ShapeDtypeStruct((HIST, DIM // 8, NW, 8 * BBLK), jnp.float32),
    compiler_params=pltpu.CompilerParams(
        use_tc_tiling_on_sc=False, needs_layout_passes=False
    ),
    scratch_types=[
        pltpu.VMEM((HIST, BBLK), jnp.int32),
        pltpu.VMEM((2, BBLK, DIM), jnp.float32),
        pltpu.VMEM((2, DIM // 8, 8 * BBLK), jnp.float32),
        pltpu.SemaphoreType.DMA,
        pltpu.SemaphoreType.DMA,
        pltpu.SemaphoreType.DMA,
        pltpu.SemaphoreType.DMA,
    ],
)
def _gather(table_hbm, xt_hbm, out_hbm, idx_v, rows_v, tbuf_v, g0, g1, w0, w1):
    w = lax.axis_index("s") * 2 + lax.axis_index("c")
    gsem = (g0, g1)
    wsem = (w0, w1)

    pltpu.sync_copy(xt_hbm.at[:, pl.ds(w * BBLK, BBLK)], idx_v)

    iotas = [lax.iota(jnp.int32, L) + gb * L for gb in range(BBLK // L)]

    def gdesc(b, u):
        return pltpu.make_async_copy(
            table_hbm.at[idx_v.at[u]], rows_v.at[b], gsem[b]
        )

    def wdesc(b, u):
        return pltpu.make_async_copy(
            tbuf_v.at[b], out_hbm.at[u, :, w], wsem[b]
        )

    def transpose(b):
        rb = rows_v.at[b]
        tb = tbuf_v.at[b]
        for j in range(DIM):
            col = jnp.full((L,), j, dtype=jnp.int32)
            for gb in range(BBLK // L):
                v = plsc.load_gather(rb, [iotas[gb], col])
                tb[j // 8, pl.ds((j % 8) * BBLK + gb * L, L)] = v

    gdesc(0, 0).start()
    gdesc(1, 1).start()

    def body(g, carry):
        for b in range(2):
            u = 2 * g + b
            gdesc(b, u).wait()

            @pl.when(g > 0)
            def _():
                wdesc(b, u - 2).wait()

            transpose(b)
            wdesc(b, u).start()

            @pl.when(g < HIST // 2 - 1)
            def _():
                gdesc(b, u + 2).start()
        return carry

    lax.fori_loop(0, HIST // 2, body, 0)

    for b in range(2):
        wdesc(b, HIST - 2 + b).wait()


def kernel(x, table):
    xt = jnp.swapaxes(x, 0, 1)
    out5 = _gather(table, xt)
    out = out5.reshape(HIST, DIM // 8, NW, 8, BBLK)
    out = out.transpose(2, 4, 0, 1, 3)
    return out.reshape(BATCH, HIST, DIM)


# speedup vs baseline: 1.7378x; 1.7378x over previous
import functools

import jax
import jax.numpy as jnp
from jax import lax
from jax.experimental import pallas as pl
from jax.experimental.pallas import tpu as pltpu
from jax.experimental.pallas import tpu_sc as plsc

VOCAB = 100000
DIM = 64
BATCH = 4096
HIST = 50

NW = 32
BBLK = BATCH // NW
L = 16

_mesh = plsc.VectorSubcoreMesh(core_axis_name="c", subcore_axis_name="s")


@functools.partial(
    pl.kernel,
    mesh=_mesh,
    out_type=jax.ShapeDtypeStruct((HIST, DIM // 8, NW, 8 * BBLK), jnp.float32),
    compiler_params=pltpu.CompilerParams(
        use_tc_tiling_on_sc=False, needs_layout_passes=False
    ),
    scratch_types=[
        pltpu.VMEM((HIST, BBLK), jnp.int32),
        pltpu.VMEM((2, BBLK, DIM), jnp.float32),
        pltpu.VMEM((2, DIM * BBLK), jnp.float32),
        pltpu.SemaphoreType.DMA,
        pltpu.SemaphoreType.DMA,
        pltpu.SemaphoreType.DMA,
        pltpu.SemaphoreType.DMA,
    ],
)
def _gather(table_hbm, xt_hbm, out_hbm, idx_v, rows_v, tbuf_v, g0, g1, w0, w1):
    w = lax.axis_index("s") * 2 + lax.axis_index("c")
    gsem = (g0, g1)
    wsem = (w0, w1)

    pltpu.sync_copy(xt_hbm.at[:, pl.ds(w * BBLK, BBLK)], idx_v)

    iotas = [lax.iota(jnp.int32, L) + gb * L for gb in range(BBLK // L)]

    def gdesc(b, u):
        return pltpu.make_async_copy(
            table_hbm.at[idx_v.at[u]], rows_v.at[b], gsem[b]
        )

    def wdescs(b, u):
        return [
            pltpu.make_async_copy(
                tbuf_v.at[b, pl.ds(dh * 8 * BBLK, 8 * BBLK)],
                out_hbm.at[u, dh, w],
                wsem[b],
            )
            for dh in range(DIM // 8)
        ]

    def transpose(b):
        rb = rows_v.at[b]
        tb = tbuf_v.at[b]

        @plsc.parallel_loop(0, DIM, unroll=8)
        def _(j):
            col = jnp.full((L,), j, dtype=jnp.int32)
            base = (j >> 3) * (8 * BBLK) + (j & 7) * BBLK
            for gb in range(BBLK // L):
                v = plsc.load_gather(rb, [iotas[gb], col])
                tb[pl.ds(base + gb * L, L)] = v

    gdesc(0, 0).start()
    gdesc(1, 1).start()

    def body(g, carry):
        for b in range(2):
            u = 2 * g + b
            gdesc(b, u).wait()

            @pl.when(g > 0)
            def _():
                for d in wdescs(b, u - 2):
                    d.wait()

            transpose(b)
            for d in wdescs(b, u):
                d.start()

            @pl.when(g < HIST // 2 - 1)
            def _():
                gdesc(b, u + 2).start()
        return carry

    lax.fori_loop(0, HIST // 2, body, 0)

    for b in range(2):
        for d in wdescs(b, HIST - 2 + b):
            d.wait()


def kernel(x, table):
    xt = jnp.swapaxes(x, 0, 1)
    out5 = _gather(table, xt)
    out = out5.reshape(HIST, DIM // 8, NW, 8, BBLK)
    out = out.transpose(2, 4, 0, 1, 3)
    return out.reshape(BATCH, HIST, DIM)


# speedup vs baseline: 2.9410x; 1.6924x over previous
import functools

import jax
import jax.numpy as jnp
from jax import lax
from jax.experimental import pallas as pl
from jax.experimental.pallas import tpu as pltpu
from jax.experimental.pallas import tpu_sc as plsc

VOCAB = 100000
DIM = 64
BATCH = 4096
HIST = 50

NW = 32
BBLK = BATCH // NW
L = 16

_mesh = plsc.VectorSubcoreMesh(core_axis_name="c", subcore_axis_name="s")


@functools.partial(
    pl.kernel,
    mesh=_mesh,
    out_type=jax.ShapeDtypeStruct((HIST, DIM // 8, NW, 8 * BBLK), jnp.float32),
    compiler_params=pltpu.CompilerParams(
        use_tc_tiling_on_sc=False, needs_layout_passes=False
    ),
    scratch_types=[
        pltpu.VMEM((HIST, BBLK), jnp.int32),
        pltpu.VMEM((2, BBLK, DIM), jnp.float32),
        pltpu.VMEM((2, DIM * BBLK), jnp.float32),
        pltpu.SemaphoreType.DMA,
        pltpu.SemaphoreType.DMA,
        pltpu.SemaphoreType.DMA,
        pltpu.SemaphoreType.DMA,
    ],
)
def _gather(table_hbm, xt_hbm, out_hbm, idx_v, rows_v, tbuf_v, g0, g1, w0, w1):
    w = lax.axis_index("s") * 2 + lax.axis_index("c")
    gsem = (g0, g1)
    wsem = (w0, w1)

    pltpu.sync_copy(xt_hbm.at[:, pl.ds(w * BBLK, BBLK)], idx_v)

    iota = lax.iota(jnp.int32, L)
    iotas = [iota + gb * L for gb in range(BBLK // L)]

    def gdesc(b, u):
        return pltpu.make_async_copy(
            table_hbm.at[idx_v.at[u]], rows_v.at[b], gsem[b]
        )

    def wdescs(b, u):
        return [
            pltpu.make_async_copy(
                tbuf_v.at[b, pl.ds(dh * 8 * BBLK, 8 * BBLK)],
                out_hbm.at[u, dh, w],
                wsem[b],
            )
            for dh in range(DIM // 8)
        ]

    def transpose(b):
        rb = rows_v.at[b]
        tb = tbuf_v.at[b]

        @plsc.parallel_loop(0, L, unroll=4)
        def _(jj):
            c = (jj + iota) & (L - 1)
            bvec = ((c >> 3) << 10) + ((c & 7) << 7) + iota
            for jb in range(DIM // L):
                colv = c + jb * L
                for gb in range(BBLK // L):
                    v = plsc.load_gather(rb, [iotas[gb], colv])
                    plsc.store_scatter(
                        tb, [bvec + (jb * (L * BBLK) + gb * L)], v
                    )

    gdesc(0, 0).start()
    gdesc(1, 1).start()

    def body(g, carry):
        for b in range(2):
            u = 2 * g + b
            gdesc(b, u).wait()

            @pl.when(g > 0)
            def _():
                for d in wdescs(b, u - 2):
                    d.wait()

            transpose(b)
            for d in wdescs(b, u):
                d.start()

            @pl.when(g < HIST // 2 - 1)
            def _():
                gdesc(b, u + 2).start()
        return carry

    lax.fori_loop(0, HIST // 2, body, 0)

    for b in range(2):
        for d in wdescs(b, HIST - 2 + b):
            d.wait()


def kernel(x, table):
    xt = jnp.swapaxes(x, 0, 1)
    out5 = _gather(table, xt)
    out = out5.reshape(HIST, DIM // 8, NW, 8, BBLK)
    out = out.transpose(2, 4, 0, 1, 3)
    return out.reshape(BATCH, HIST, DIM)


# speedup vs baseline: 3.2645x; 1.1100x over previous
import functools

import jax
import jax.numpy as jnp
from jax import lax
from jax.experimental import pallas as pl
from jax.experimental.pallas import tpu as pltpu
from jax.experimental.pallas import tpu_sc as plsc

VOCAB = 100000
DIM = 64
BATCH = 4096
HIST = 50

NW = 32
BBLK = BATCH // NW
L = 16

_mesh = plsc.VectorSubcoreMesh(core_axis_name="c", subcore_axis_name="s")


@functools.partial(
    pl.kernel,
    mesh=_mesh,
    out_type=jax.ShapeDtypeStruct((HIST, DIM // 8, NW, 8 * BBLK), jnp.float32),
    compiler_params=pltpu.CompilerParams(
        use_tc_tiling_on_sc=False, needs_layout_passes=False
    ),
    scratch_types=[
        pltpu.VMEM((HIST, BBLK), jnp.int32),
        pltpu.VMEM((2, BBLK, DIM), jnp.float32),
        pltpu.VMEM((2, DIM // 8, 8 * BBLK), jnp.float32),
        pltpu.SemaphoreType.DMA,
        pltpu.SemaphoreType.DMA,
        pltpu.SemaphoreType.DMA,
        pltpu.SemaphoreType.DMA,
    ],
)
def _gather(table_hbm, xt_hbm, out_hbm, idx_v, rows_v, tbuf_v, g0, g1, w0, w1):
    w = lax.axis_index("s") * 2 + lax.axis_index("c")
    gsem = (g0, g1)
    wsem = (w0, w1)

    pltpu.sync_copy(xt_hbm.at[:, pl.ds(w * BBLK, BBLK)], idx_v)

    iota = lax.iota(jnp.int32, L)
    iotas = [iota + gb * L for gb in range(BBLK // L)]

    def gdesc(b, u):
        return pltpu.make_async_copy(
            table_hbm.at[idx_v.at[u]], rows_v.at[b], gsem[b]
        )

    def wdescs(b, u):
        return [
            pltpu.make_async_copy(
                tbuf_v.at[b], out_hbm.at[u, :, w], wsem[b]
            )
        ]

    def transpose(b):
        rb = rows_v.at[b]
        tb = tbuf_v.at[b]

        @plsc.parallel_loop(0, L, unroll=8)
        def _(jj):
            c = (jj + iota) & (L - 1)
            dhv = c >> 3
            innerb = ((c & 7) << 7) + iota
            for jb in range(DIM // L):
                colv = c + jb * L
                dh_jb = dhv + 2 * jb
                for gb in range(BBLK // L):
                    v = plsc.load_gather(rb, [iotas[gb], colv])
                    plsc.store_scatter(tb, [dh_jb, innerb + gb * L], v)

    gdesc(0, 0).start()
    gdesc(1, 1).start()

    def body(g, carry):
        for b in range(2):
            u = 2 * g + b
            gdesc(b, u).wait()

            @pl.when(g > 0)
            def _():
                for d in wdescs(b, u - 2):
                    d.wait()

            transpose(b)
            for d in wdescs(b, u):
                d.start()

            @pl.when(g < HIST // 2 - 1)
            def _():
                gdesc(b, u + 2).start()
        return carry

    lax.fori_loop(0, HIST // 2, body, 0)

    for b in range(2):
        for d in wdescs(b, HIST - 2 + b):
            d.wait()


def kernel(x, table):
    xt = jnp.swapaxes(x, 0, 1)
    out5 = _gather(table, xt)
    out = out5.reshape(HIST, DIM // 8, NW, 8, BBLK)
    out = out.transpose(2, 4, 0, 1, 3)
    return out.reshape(BATCH, HIST, DIM)
